# Initial kernel scaffold; baseline (speedup 1.0000x reference)
import jax
import jax.numpy as jnp
from jax.experimental import pallas as pl


def kernel(x, edge_index, W1, W2, W3):
    raise NotImplementedError("write your pallas kernel here")


# speedup vs baseline: 14.5682x; 14.5682x over previous
import functools

import jax
import jax.numpy as jnp
from jax import lax
from jax.experimental import pallas as pl
from jax.experimental.pallas import tpu as pltpu
from jax.experimental.pallas import tpu_sc as plsc

N = 10000
E = 320000
D = 128

NC = 2
NS = 16
NW = NC * NS
G = 128
NG = E // G
ITERS = -(-NG // NW)
NPAD = 10240
RPS = NPAD // NS


def _edge_mesh():
  return plsc.VectorSubcoreMesh(core_axis_name="c", subcore_axis_name="s")


def _make_sc_edge(width):
  def body(table, src2d, dst2d, zeros, out, shared, src_v, dst_v, rows_v, sem):
    c = lax.axis_index("c")
    s = lax.axis_index("s")
    wid = s * NC + c

    pltpu.sync_copy(zeros, shared.at[pl.ds(s * RPS, RPS)])
    plsc.subcore_barrier()

    def step(i, carry):
      g = wid + NW * i

      @pl.when(g < NG)
      def _():
        pltpu.sync_copy(src2d.at[g], src_v)
        pltpu.sync_copy(dst2d.at[g], dst_v)
        pltpu.async_copy(table.at[src_v], rows_v, sem).wait()
        pltpu.sync_copy(rows_v, shared.at[dst_v], add=True)

      return carry

    lax.fori_loop(0, ITERS, step, 0)
    plsc.subcore_barrier()

    pltpu.sync_copy(
        shared.at[pl.ds(s * RPS, RPS)],
        out.at[pl.ds(c * NPAD + s * RPS, RPS)],
    )

  return pl.kernel(
      body,
      out_type=jax.ShapeDtypeStruct((NC * NPAD, width), jnp.float32),
      mesh=_edge_mesh(),
      scratch_types=[
          pltpu.VMEM_SHARED((NPAD, width), jnp.float32),
          pltpu.VMEM((G,), jnp.int32),
          pltpu.VMEM((G,), jnp.int32),
          pltpu.VMEM((G, width), jnp.float32),
          pltpu.SemaphoreType.DMA,
      ],
      compiler_params=pltpu.CompilerParams(use_tc_tiling_on_sc=False),
  )


def _sc_degree(dst2d, zeros):
  def body(dst2d, zeros, out, shared, dst_v, ones_v, _sem):
    c = lax.axis_index("c")
    s = lax.axis_index("s")
    wid = s * NC + c

    for k in range(8):
      ones_v[pl.ds(k * 16, 16), :] = jnp.ones((16, 8), jnp.float32)
    pltpu.sync_copy(zeros, shared.at[pl.ds(s * RPS, RPS)])
    plsc.subcore_barrier()

    def step(i, carry):
      g = wid + NW * i

      @pl.when(g < NG)
      def _():
        pltpu.sync_copy(dst2d.at[g], dst_v)
        pltpu.sync_copy(ones_v, shared.at[dst_v], add=True)

      return carry

    lax.fori_loop(0, ITERS, step, 0)
    plsc.subcore_barrier()
    pltpu.sync_copy(
        shared.at[pl.ds(s * RPS, RPS)],
        out.at[pl.ds(c * NPAD + s * RPS, RPS)],
    )

  return pl.kernel(
      body,
      out_type=jax.ShapeDtypeStruct((NC * NPAD, 8), jnp.float32),
      mesh=_edge_mesh(),
      scratch_types=[
          pltpu.VMEM_SHARED((NPAD, 8), jnp.float32),
          pltpu.VMEM((G,), jnp.int32),
          pltpu.VMEM((G, 8), jnp.float32),
          pltpu.SemaphoreType.DMA,
      ],
      compiler_params=pltpu.CompilerParams(use_tc_tiling_on_sc=False),
  )(dst2d, zeros)


_RB = 2000
_GRID = N // _RB


def _row_spec(w):
  return pl.BlockSpec((_RB, w), lambda i: (i, 0))


def _full_spec(shape):
  return pl.BlockSpec(shape, lambda i: (0, 0))


def _tc1_body(d0_ref, d1_ref, x_ref, w1_ref, isd_ref, hw_ref):
  deg = d0_ref[...] + d1_ref[...] + 1.0
  isd = lax.rsqrt(deg)
  isd_ref[...] = isd
  hw = jnp.dot(x_ref[...], w1_ref[...], preferred_element_type=jnp.float32)
  hw_ref[...] = hw * isd


def _tc1(d0, d1, x, w1p):
  return pl.pallas_call(
      _tc1_body,
      grid=(_GRID,),
      in_specs=[_row_spec(1), _row_spec(1), _row_spec(D), _full_spec((D, 32))],
      out_specs=[_row_spec(1), _row_spec(32)],
      out_shape=[
          jax.ShapeDtypeStruct((N, 1), jnp.float32),
          jax.ShapeDtypeStruct((N, 32), jnp.float32),
      ],
  )(d0, d1, x, w1p)


def _tc2_body(s0_ref, s1_ref, hw_ref, isd_ref, out_ref):
  isd = isd_ref[...]
  agg = isd * (s0_ref[...] + s1_ref[...] + hw_ref[...])
  out_ref[...] = jnp.maximum(agg, 0.0) * isd


def _tc2(s0, s1, hw, isd):
  return pl.pallas_call(
      _tc2_body,
      grid=(_GRID,),
      in_specs=[_row_spec(32), _row_spec(32), _row_spec(32), _row_spec(1)],
      out_specs=_row_spec(32),
      out_shape=jax.ShapeDtypeStruct((N, 32), jnp.float32),
  )(s0, s1, hw, isd)


def _tc3_body(s0_ref, s1_ref, h1p_ref, isd_ref, w2_ref, w3_ref, out_ref):
  isd = isd_ref[...]
  t = isd * (s0_ref[...] + s1_ref[...] + h1p_ref[...])
  h2 = jnp.maximum(
      jnp.dot(t, w2_ref[...], preferred_element_type=jnp.float32), 0.0)
  hw3 = jnp.dot(h2, w3_ref[...], preferred_element_type=jnp.float32)
  out_ref[...] = hw3 * isd


def _tc3(s0, s1, h1p, isd, w2p, w3p):
  return pl.pallas_call(
      _tc3_body,
      grid=(_GRID,),
      in_specs=[
          _row_spec(32), _row_spec(32), _row_spec(32), _row_spec(1),
          _full_spec((32, 40)), _full_spec((40, 8)),
      ],
      out_specs=_row_spec(8),
      out_shape=jax.ShapeDtypeStruct((N, 8), jnp.float32),
  )(s0, s1, h1p, isd, w2p, w3p)


def _tc4_body(s0_ref, s1_ref, hw_ref, isd_ref, out_ref):
  out_ref[...] = isd_ref[...] * (s0_ref[...] + s1_ref[...] + hw_ref[...])


def _tc4(s0, s1, hw, isd):
  return pl.pallas_call(
      _tc4_body,
      grid=(_GRID,),
      in_specs=[_row_spec(8), _row_spec(8), _row_spec(8), _row_spec(1)],
      out_specs=_row_spec(8),
      out_shape=jax.ShapeDtypeStruct((N, 8), jnp.float32),
  )(s0, s1, hw, isd)


_sc_edge32 = _make_sc_edge(32)
_sc_edge8 = _make_sc_edge(8)


@jax.jit
def kernel(x, edge_index, W1, W2, W3):
  src2d = edge_index[0].reshape(NG, G)
  dst2d = edge_index[1].reshape(NG, G)
  w1p = jnp.pad(W1, ((0, 0), (0, 32 - W1.shape[1])))
  w2p = jnp.pad(W2, ((0, 32 - W2.shape[0]), (0, 0)))
  w3p = jnp.pad(W3, ((0, 0), (0, 8 - W3.shape[1])))
  zeros8 = jnp.zeros((RPS, 8), jnp.float32)
  zeros32 = jnp.zeros((RPS, 32), jnp.float32)

  degp = _sc_degree(dst2d, zeros8)
  d0 = degp[:N, :1]
  d1 = degp[NPAD:NPAD + N, :1]

  isd, hw1p = _tc1(d0, d1, x, w1p)
  s1 = _sc_edge32(hw1p, src2d, dst2d, zeros32)
  h1p = _tc2(s1[:N], s1[NPAD:NPAD + N], hw1p, isd)
  s2 = _sc_edge32(h1p, src2d, dst2d, zeros32)
  hw3p = _tc3(s2[:N], s2[NPAD:NPAD + N], h1p, isd, w2p, w3p)
  s3 = _sc_edge8(hw3p, src2d, dst2d, zeros8)
  out8 = _tc4(s3[:N], s3[NPAD:NPAD + N], hw3p, isd)
  return out8[:, :2]


# speedup vs baseline: 20.4949x; 1.4068x over previous
import functools

import jax
import jax.numpy as jnp
from jax import lax
from jax.experimental import pallas as pl
from jax.experimental.pallas import tpu as pltpu
from jax.experimental.pallas import tpu_sc as plsc

N = 10000
E = 320000
D = 128

NC = 2
NS = 16
NW = NC * NS
G = 128
K = 10
EPAD = 327680
NGP = EPAD // G
NCH = NGP // K
T = NCH // NW
NPAD = 10240
RPS = NPAD // NS


def _edge_mesh():
  return plsc.VectorSubcoreMesh(core_axis_name="c", subcore_axis_name="s")


def _make_sc_edge(width):
  def body(table, src2d, dst2d, zeros, out, shared, src_v, dst_v, rows_v,
           sem_i, sem_g, sem_s):
    c = lax.axis_index("c")
    s = lax.axis_index("s")
    wid = s * NC + c
    base = wid * T

    pltpu.sync_copy(zeros, shared.at[pl.ds(s * RPS, RPS)])
    plsc.subcore_barrier()

    idx_d, gat_d, sct_d = {}, {}, {}

    def start_idx(t):
      slot = t % 3
      ch = base + t
      idx_d[t] = (
          pltpu.async_copy(src2d.at[pl.ds(ch * K, K)], src_v.at[slot], sem_i),
          pltpu.async_copy(dst2d.at[pl.ds(ch * K, K)], dst_v.at[slot], sem_i),
      )

    start_idx(0)
    for t in range(T):
      islot = t % 3
      rslot = t % 2
      if t >= 2:
        for d in sct_d.pop(t - 2):
          d.wait()
      for d in idx_d.pop(t):
        d.wait()
      gat_d[t] = [
          pltpu.async_copy(table.at[src_v.at[islot, j]], rows_v.at[rslot, j],
                           sem_g)
          for j in range(K)
      ]
      if t + 1 < T:
        start_idx(t + 1)
      for d in gat_d.pop(t):
        d.wait()
      sct_d[t] = [
          pltpu.async_copy(rows_v.at[rslot, j], shared.at[dst_v.at[islot, j]],
                           sem_s, add=True)
          for j in range(K)
      ]
    for t in (T - 2, T - 1):
      for d in sct_d.pop(t):
        d.wait()

    plsc.subcore_barrier()

    pltpu.sync_copy(
        shared.at[pl.ds(s * RPS, RPS)],
        out.at[pl.ds(c * NPAD + s * RPS, RPS)],
    )

  return pl.kernel(
      body,
      out_type=jax.ShapeDtypeStruct((NC * NPAD, width), jnp.float32),
      mesh=_edge_mesh(),
      scratch_types=[
          pltpu.VMEM_SHARED((NPAD, width), jnp.float32),
          pltpu.VMEM((3, K, G), jnp.int32),
          pltpu.VMEM((3, K, G), jnp.int32),
          pltpu.VMEM((2, K, G, width), jnp.float32),
          pltpu.SemaphoreType.DMA,
          pltpu.SemaphoreType.DMA,
          pltpu.SemaphoreType.DMA,
      ],
      compiler_params=pltpu.CompilerParams(use_tc_tiling_on_sc=False),
  )


def _sc_degree(dst2d, zeros):
  def body(dst2d, zeros, out, shared, dst_v, ones_v, sem_i, sem_s):
    c = lax.axis_index("c")
    s = lax.axis_index("s")
    wid = s * NC + c
    base = wid * T

    for k in range(8):
      ones_v[pl.ds(k * 16, 16), :] = jnp.ones((16, 8), jnp.float32)
    pltpu.sync_copy(zeros, shared.at[pl.ds(s * RPS, RPS)])
    plsc.subcore_barrier()

    idx_d, sct_d = {}, {}

    def start_idx(t):
      ch = base + t
      idx_d[t] = pltpu.async_copy(
          dst2d.at[pl.ds(ch * K, K)], dst_v.at[t % 3], sem_i)

    start_idx(0)
    for t in range(T):
      islot = t % 3
      if t >= 2:
        for d in sct_d.pop(t - 2):
          d.wait()
      idx_d.pop(t).wait()
      if t + 1 < T:
        start_idx(t + 1)
      sct_d[t] = [
          pltpu.async_copy(ones_v, shared.at[dst_v.at[islot, j]], sem_s,
                           add=True)
          for j in range(K)
      ]
    for t in (T - 2, T - 1):
      for d in sct_d.pop(t):
        d.wait()

    plsc.subcore_barrier()
    pltpu.sync_copy(
        shared.at[pl.ds(s * RPS, RPS)],
        out.at[pl.ds(c * NPAD + s * RPS, RPS)],
    )

  return pl.kernel(
      body,
      out_type=jax.ShapeDtypeStruct((NC * NPAD, 8), jnp.float32),
      mesh=_edge_mesh(),
      scratch_types=[
          pltpu.VMEM_SHARED((NPAD, 8), jnp.float32),
          pltpu.VMEM((3, K, G), jnp.int32),
          pltpu.VMEM((G, 8), jnp.float32),
          pltpu.SemaphoreType.DMA,
          pltpu.SemaphoreType.DMA,
      ],
      compiler_params=pltpu.CompilerParams(use_tc_tiling_on_sc=False),
  )(dst2d, zeros)


_RB = 2000
_GRID = N // _RB


def _row_spec(w):
  return pl.BlockSpec((_RB, w), lambda i: (i, 0))


def _full_spec(shape):
  return pl.BlockSpec(shape, lambda i: (0, 0))


def _tc1_body(d0_ref, d1_ref, x_ref, w1_ref, isd_ref, hw_ref):
  deg = d0_ref[...] + d1_ref[...] + 1.0
  isd = lax.rsqrt(deg)
  isd_ref[...] = isd
  hw = jnp.dot(x_ref[...], w1_ref[...], preferred_element_type=jnp.float32)
  hw_ref[...] = hw * isd


def _tc1(d0, d1, x, w1p):
  return pl.pallas_call(
      _tc1_body,
      grid=(_GRID,),
      in_specs=[_row_spec(1), _row_spec(1), _row_spec(D), _full_spec((D, 32))],
      out_specs=[_row_spec(1), _row_spec(32)],
      out_shape=[
          jax.ShapeDtypeStruct((N, 1), jnp.float32),
          jax.ShapeDtypeStruct((N, 32), jnp.float32),
      ],
  )(d0, d1, x, w1p)


def _tc2_body(s0_ref, s1_ref, hw_ref, isd_ref, out_ref):
  isd = isd_ref[...]
  agg = isd * (s0_ref[...] + s1_ref[...] + hw_ref[...])
  out_ref[...] = jnp.maximum(agg, 0.0) * isd


def _tc2(s0, s1, hw, isd):
  return pl.pallas_call(
      _tc2_body,
      grid=(_GRID,),
      in_specs=[_row_spec(32), _row_spec(32), _row_spec(32), _row_spec(1)],
      out_specs=_row_spec(32),
      out_shape=jax.ShapeDtypeStruct((N, 32), jnp.float32),
  )(s0, s1, hw, isd)


def _tc3_body(s0_ref, s1_ref, h1p_ref, isd_ref, w2_ref, w3_ref, out_ref):
  isd = isd_ref[...]
  t = isd * (s0_ref[...] + s1_ref[...] + h1p_ref[...])
  h2 = jnp.maximum(
      jnp.dot(t, w2_ref[...], preferred_element_type=jnp.float32), 0.0)
  hw3 = jnp.dot(h2, w3_ref[...], preferred_element_type=jnp.float32)
  out_ref[...] = hw3 * isd


def _tc3(s0, s1, h1p, isd, w2p, w3p):
  return pl.pallas_call(
      _tc3_body,
      grid=(_GRID,),
      in_specs=[
          _row_spec(32), _row_spec(32), _row_spec(32), _row_spec(1),
          _full_spec((32, 40)), _full_spec((40, 8)),
      ],
      out_specs=_row_spec(8),
      out_shape=jax.ShapeDtypeStruct((N, 8), jnp.float32),
  )(s0, s1, h1p, isd, w2p, w3p)


def _tc4_body(s0_ref, s1_ref, hw_ref, isd_ref, out_ref):
  out_ref[...] = isd_ref[...] * (s0_ref[...] + s1_ref[...] + hw_ref[...])


def _tc4(s0, s1, hw, isd):
  return pl.pallas_call(
      _tc4_body,
      grid=(_GRID,),
      in_specs=[_row_spec(8), _row_spec(8), _row_spec(8), _row_spec(1)],
      out_specs=_row_spec(8),
      out_shape=jax.ShapeDtypeStruct((N, 8), jnp.float32),
  )(s0, s1, hw, isd)


_sc_edge32 = _make_sc_edge(32)
_sc_edge8 = _make_sc_edge(8)


@jax.jit
def kernel(x, edge_index, W1, W2, W3):
  pad = EPAD - E
  src2d = jnp.concatenate(
      [edge_index[0], jnp.zeros((pad,), jnp.int32)]).reshape(NGP, G)
  dst2d = jnp.concatenate(
      [edge_index[1], jnp.full((pad,), N, jnp.int32)]).reshape(NGP, G)
  w1p = jnp.pad(W1, ((0, 0), (0, 32 - W1.shape[1])))
  w2p = jnp.pad(W2, ((0, 32 - W2.shape[0]), (0, 0)))
  w3p = jnp.pad(W3, ((0, 0), (0, 8 - W3.shape[1])))
  zeros8 = jnp.zeros((RPS, 8), jnp.float32)
  zeros32 = jnp.zeros((RPS, 32), jnp.float32)

  degp = _sc_degree(dst2d, zeros8)
  d0 = degp[:N, :1]
  d1 = degp[NPAD:NPAD + N, :1]

  isd, hw1p = _tc1(d0, d1, x, w1p)
  s1 = _sc_edge32(hw1p, src2d, dst2d, zeros32)
  h1p = _tc2(s1[:N], s1[NPAD:NPAD + N], hw1p, isd)
  s2 = _sc_edge32(h1p, src2d, dst2d, zeros32)
  hw3p = _tc3(s2[:N], s2[NPAD:NPAD + N], h1p, isd, w2p, w3p)
  s3 = _sc_edge8(hw3p, src2d, dst2d, zeros8)
  out8 = _tc4(s3[:N], s3[NPAD:NPAD + N], hw3p, isd)
  return out8[:, :2]


# speedup vs baseline: 22.9294x; 1.1188x over previous
import functools

import jax
import jax.numpy as jnp
from jax import lax
from jax.experimental import pallas as pl
from jax.experimental.pallas import tpu as pltpu
from jax.experimental.pallas import tpu_sc as plsc

N = 10000
E = 320000
D = 128

NC = 2
NS = 16
NW = NC * NS
G = 128
K = 10
EPAD = 327680
NGP = EPAD // G
NCH = NGP // K
T = NCH // NW
NPAD = 10240
RPS = NPAD // NS


def _edge_mesh():
  return plsc.VectorSubcoreMesh(core_axis_name="c", subcore_axis_name="s")


def _make_sc_edge(width):
  def body(table, src2d, dst2d, zeros, out, shared, src_v, dst_v, rows_v,
           sem_i, sem_g, sem_s):
    c = lax.axis_index("c")
    s = lax.axis_index("s")
    wid = s * NC + c
    base = wid * T

    pltpu.sync_copy(zeros, shared.at[pl.ds(s * RPS, RPS)])
    plsc.subcore_barrier()

    idx_d, gat_d, sct_d = {}, {}, {}

    def start_idx(t):
      slot = t % 3
      ch = base + t
      idx_d[t] = (
          pltpu.async_copy(src2d.at[pl.ds(ch * K, K)], src_v.at[slot], sem_i),
          pltpu.async_copy(dst2d.at[pl.ds(ch * K, K)], dst_v.at[slot], sem_i),
      )

    start_idx(0)
    for t in range(T):
      islot = t % 3
      rslot = t % 2
      if t >= 2:
        for d in sct_d.pop(t - 2):
          d.wait()
      for d in idx_d.pop(t):
        d.wait()
      gat_d[t] = [
          pltpu.async_copy(table.at[src_v.at[islot, j]], rows_v.at[rslot, j],
                           sem_g)
          for j in range(K)
      ]
      if t + 1 < T:
        start_idx(t + 1)
      for d in gat_d.pop(t):
        d.wait()
      sct_d[t] = [
          pltpu.async_copy(rows_v.at[rslot, j], shared.at[dst_v.at[islot, j]],
                           sem_s, add=True)
          for j in range(K)
      ]
    for t in (T - 2, T - 1):
      for d in sct_d.pop(t):
        d.wait()

    plsc.subcore_barrier()

    pltpu.sync_copy(
        shared.at[pl.ds(s * RPS, RPS)],
        out.at[pl.ds(c * NPAD + s * RPS, RPS)],
    )

  return pl.kernel(
      body,
      out_type=jax.ShapeDtypeStruct((NC * NPAD, width), jnp.float32),
      mesh=_edge_mesh(),
      scratch_types=[
          pltpu.VMEM_SHARED((NPAD, width), jnp.float32),
          pltpu.VMEM((3, K, G), jnp.int32),
          pltpu.VMEM((3, K, G), jnp.int32),
          pltpu.VMEM((2, K, G, width), jnp.float32),
          pltpu.SemaphoreType.DMA,
          pltpu.SemaphoreType.DMA,
          pltpu.SemaphoreType.DMA,
      ],
      compiler_params=pltpu.CompilerParams(use_tc_tiling_on_sc=False),
  )


def _sc_degree(dst2d, zeros):
  def body(dst2d, zeros, out, shared, dst_v, ones_v, sem_i, sem_s):
    c = lax.axis_index("c")
    s = lax.axis_index("s")
    wid = s * NC + c
    base = wid * T

    for k in range(8):
      ones_v[pl.ds(k * 16, 16), :] = jnp.ones((16, 8), jnp.float32)
    pltpu.sync_copy(zeros, shared.at[pl.ds(s * RPS, RPS)])
    plsc.subcore_barrier()

    idx_d, sct_d = {}, {}

    def start_idx(t):
      ch = base + t
      idx_d[t] = pltpu.async_copy(
          dst2d.at[pl.ds(ch * K, K)], dst_v.at[t % 3], sem_i)

    start_idx(0)
    for t in range(T):
      islot = t % 3
      if t >= 2:
        for d in sct_d.pop(t - 2):
          d.wait()
      idx_d.pop(t).wait()
      if t + 1 < T:
        start_idx(t + 1)
      sct_d[t] = [
          pltpu.async_copy(ones_v, shared.at[dst_v.at[islot, j]], sem_s,
                           add=True)
          for j in range(K)
      ]
    for t in (T - 2, T - 1):
      for d in sct_d.pop(t):
        d.wait()

    plsc.subcore_barrier()
    pltpu.sync_copy(
        shared.at[pl.ds(s * RPS, RPS)],
        out.at[pl.ds(c * NPAD + s * RPS, RPS)],
    )

  return pl.kernel(
      body,
      out_type=jax.ShapeDtypeStruct((NC * NPAD, 8), jnp.float32),
      mesh=_edge_mesh(),
      scratch_types=[
          pltpu.VMEM_SHARED((NPAD, 8), jnp.float32),
          pltpu.VMEM((3, K, G), jnp.int32),
          pltpu.VMEM((G, 8), jnp.float32),
          pltpu.SemaphoreType.DMA,
          pltpu.SemaphoreType.DMA,
      ],
      compiler_params=pltpu.CompilerParams(use_tc_tiling_on_sc=False),
  )(dst2d, zeros)


_RB = 2000
_GRID = N // _RB


def _row_spec(w):
  return pl.BlockSpec((_RB, w), lambda i: (i, 0))


def _full_spec(shape):
  return pl.BlockSpec(shape, lambda i: (0, 0))


def _tc1_body(d0_ref, d1_ref, x_ref, w1_ref, isd_ref, hw_ref):
  deg = d0_ref[...] + d1_ref[...] + 1.0
  isd = lax.rsqrt(deg)
  isd_ref[...] = isd
  hw = jnp.dot(x_ref[...], w1_ref[...], preferred_element_type=jnp.float32)
  hw_ref[...] = hw * isd


def _tc1(d0, d1, x, w1p):
  return pl.pallas_call(
      _tc1_body,
      grid=(_GRID,),
      in_specs=[_row_spec(1), _row_spec(1), _row_spec(D), _full_spec((D, 24))],
      out_specs=[_row_spec(1), _row_spec(24)],
      out_shape=[
          jax.ShapeDtypeStruct((N, 1), jnp.float32),
          jax.ShapeDtypeStruct((N, 24), jnp.float32),
      ],
  )(d0, d1, x, w1p)


def _tc2_body(s0_ref, s1_ref, hw_ref, isd_ref, out_ref):
  isd = isd_ref[...]
  agg = isd * (s0_ref[...] + s1_ref[...] + hw_ref[...])
  out_ref[...] = jnp.maximum(agg, 0.0) * isd


def _tc2(s0, s1, hw, isd):
  return pl.pallas_call(
      _tc2_body,
      grid=(_GRID,),
      in_specs=[_row_spec(24), _row_spec(24), _row_spec(24), _row_spec(1)],
      out_specs=_row_spec(24),
      out_shape=jax.ShapeDtypeStruct((N, 24), jnp.float32),
  )(s0, s1, hw, isd)


def _tc3_body(s0_ref, s1_ref, h1p_ref, isd_ref, w2_ref, w3_ref, out_ref):
  isd = isd_ref[...]
  t = isd * (s0_ref[...] + s1_ref[...] + h1p_ref[...])
  h2 = jnp.maximum(
      jnp.dot(t, w2_ref[...], preferred_element_type=jnp.float32), 0.0)
  hw3 = jnp.dot(h2, w3_ref[...], preferred_element_type=jnp.float32)
  out_ref[...] = hw3 * isd


def _tc3(s0, s1, h1p, isd, w2p, w3p):
  return pl.pallas_call(
      _tc3_body,
      grid=(_GRID,),
      in_specs=[
          _row_spec(24), _row_spec(24), _row_spec(24), _row_spec(1),
          _full_spec((24, 40)), _full_spec((40, 8)),
      ],
      out_specs=_row_spec(8),
      out_shape=jax.ShapeDtypeStruct((N, 8), jnp.float32),
  )(s0, s1, h1p, isd, w2p, w3p)


def _tc4_body(s0_ref, s1_ref, hw_ref, isd_ref, out_ref):
  out_ref[...] = isd_ref[...] * (s0_ref[...] + s1_ref[...] + hw_ref[...])


def _tc4(s0, s1, hw, isd):
  return pl.pallas_call(
      _tc4_body,
      grid=(_GRID,),
      in_specs=[_row_spec(8), _row_spec(8), _row_spec(8), _row_spec(1)],
      out_specs=_row_spec(8),
      out_shape=jax.ShapeDtypeStruct((N, 8), jnp.float32),
  )(s0, s1, hw, isd)


_sc_edge24 = _make_sc_edge(24)
_sc_edge8 = _make_sc_edge(8)


@jax.jit
def kernel(x, edge_index, W1, W2, W3):
  pad = EPAD - E
  src2d = jnp.concatenate(
      [edge_index[0], jnp.zeros((pad,), jnp.int32)]).reshape(NGP, G)
  dst2d = jnp.concatenate(
      [edge_index[1], jnp.full((pad,), N, jnp.int32)]).reshape(NGP, G)
  w1p = jnp.pad(W1, ((0, 0), (0, 4)))
  w2p = jnp.pad(W2, ((0, 4), (0, 0)))
  w3p = jnp.pad(W3, ((0, 0), (0, 6)))
  zeros8 = jnp.zeros((RPS, 8), jnp.float32)
  zeros24 = jnp.zeros((RPS, 24), jnp.float32)

  degp = _sc_degree(dst2d, zeros8)
  d0 = degp[:N, :1]
  d1 = degp[NPAD:NPAD + N, :1]

  isd, hw1p = _tc1(d0, d1, x, w1p)
  s1 = _sc_edge24(hw1p, src2d, dst2d, zeros24)
  h1p = _tc2(s1[:N], s1[NPAD:NPAD + N], hw1p, isd)
  s2 = _sc_edge24(h1p, src2d, dst2d, zeros24)
  hw3p = _tc3(s2[:N], s2[NPAD:NPAD + N], h1p, isd, w2p, w3p)
  s3 = _sc_edge8(hw3p, src2d, dst2d, zeros8)
  out8 = _tc4(s3[:N], s3[NPAD:NPAD + N], hw3p, isd)
  return out8[:, :2]


# speedup vs baseline: 23.4546x; 1.0229x over previous
import functools

import jax
import jax.numpy as jnp
from jax import lax
from jax.experimental import pallas as pl
from jax.experimental.pallas import tpu as pltpu
from jax.experimental.pallas import tpu_sc as plsc

N = 10000
E = 320000
D = 128

NC = 2
NS = 16
NW = NC * NS
G = 128
K = 10
EPAD = 327680
NGP = EPAD // G
NCH = NGP // K
T = NCH // NW
NPAD = 10240
RPS = NPAD // NS


def _edge_mesh():
  return plsc.VectorSubcoreMesh(core_axis_name="c", subcore_axis_name="s")


def _make_sc_edge(width):
  def body(table, src2d, dst2d, zeros, out, shared, src_v, dst_v, rows_v,
           sem_i, sem_g, sem_s):
    c = lax.axis_index("c")
    s = lax.axis_index("s")
    wid = s * NC + c
    base = wid * T

    pltpu.sync_copy(zeros, shared.at[pl.ds(s * RPS, RPS)])
    plsc.subcore_barrier()

    idx_d, gat_d, sct_d = {}, {}, {}

    def start_idx(t):
      if t >= T:
        return
      ch = base + t
      idx_d[t] = (
          pltpu.async_copy(src2d.at[pl.ds(ch * K, K)], src_v.at[t % 3], sem_i),
          pltpu.async_copy(dst2d.at[pl.ds(ch * K, K)], dst_v.at[t % 4], sem_i),
      )

    def start_gat(t):
      if t >= T:
        return
      for d in idx_d.pop(t):
        d.wait()
      gat_d[t] = [
          pltpu.async_copy(table.at[src_v.at[t % 3, j]], rows_v.at[t % 3, j],
                           sem_g)
          for j in range(K)
      ]

    start_idx(0)
    start_idx(1)
    start_gat(0)
    for t in range(T):
      if t >= 2:
        for d in sct_d.pop(t - 2):
          d.wait()
      start_idx(t + 2)
      start_gat(t + 1)
      for d in gat_d.pop(t):
        d.wait()
      sct_d[t] = [
          pltpu.async_copy(rows_v.at[t % 3, j], shared.at[dst_v.at[t % 4, j]],
                           sem_s, add=True)
          for j in range(K)
      ]
    for t in (T - 2, T - 1):
      for d in sct_d.pop(t):
        d.wait()

    plsc.subcore_barrier()

    pltpu.sync_copy(
        shared.at[pl.ds(s * RPS, RPS)],
        out.at[pl.ds(c * NPAD + s * RPS, RPS)],
    )

  return pl.kernel(
      body,
      out_type=jax.ShapeDtypeStruct((NC * NPAD, width), jnp.float32),
      mesh=_edge_mesh(),
      scratch_types=[
          pltpu.VMEM_SHARED((NPAD, width), jnp.float32),
          pltpu.VMEM((3, K, G), jnp.int32),
          pltpu.VMEM((4, K, G), jnp.int32),
          pltpu.VMEM((3, K, G, width), jnp.float32),
          pltpu.SemaphoreType.DMA,
          pltpu.SemaphoreType.DMA,
          pltpu.SemaphoreType.DMA,
      ],
      compiler_params=pltpu.CompilerParams(use_tc_tiling_on_sc=False),
  )


def _sc_degree(dst2d, zeros):
  def body(dst2d, zeros, out, shared, dst_v, ones_v, sem_i, sem_s):
    c = lax.axis_index("c")
    s = lax.axis_index("s")
    wid = s * NC + c
    base = wid * T

    for k in range(8):
      ones_v[pl.ds(k * 16, 16), :] = jnp.ones((16, 8), jnp.float32)
    pltpu.sync_copy(zeros, shared.at[pl.ds(s * RPS, RPS)])
    plsc.subcore_barrier()

    idx_d, sct_d = {}, {}

    def start_idx(t):
      ch = base + t
      idx_d[t] = pltpu.async_copy(
          dst2d.at[pl.ds(ch * K, K)], dst_v.at[t % 3], sem_i)

    start_idx(0)
    for t in range(T):
      islot = t % 3
      if t >= 2:
        for d in sct_d.pop(t - 2):
          d.wait()
      idx_d.pop(t).wait()
      if t + 1 < T:
        start_idx(t + 1)
      sct_d[t] = [
          pltpu.async_copy(ones_v, shared.at[dst_v.at[islot, j]], sem_s,
                           add=True)
          for j in range(K)
      ]
    for t in (T - 2, T - 1):
      for d in sct_d.pop(t):
        d.wait()

    plsc.subcore_barrier()
    pltpu.sync_copy(
        shared.at[pl.ds(s * RPS, RPS)],
        out.at[pl.ds(c * NPAD + s * RPS, RPS)],
    )

  return pl.kernel(
      body,
      out_type=jax.ShapeDtypeStruct((NC * NPAD, 8), jnp.float32),
      mesh=_edge_mesh(),
      scratch_types=[
          pltpu.VMEM_SHARED((NPAD, 8), jnp.float32),
          pltpu.VMEM((3, K, G), jnp.int32),
          pltpu.VMEM((G, 8), jnp.float32),
          pltpu.SemaphoreType.DMA,
          pltpu.SemaphoreType.DMA,
      ],
      compiler_params=pltpu.CompilerParams(use_tc_tiling_on_sc=False),
  )(dst2d, zeros)


_RB = 2000
_GRID = N // _RB


def _row_spec(w):
  return pl.BlockSpec((_RB, w), lambda i: (i, 0))


def _full_spec(shape):
  return pl.BlockSpec(shape, lambda i: (0, 0))


def _tc1_body(d0_ref, d1_ref, x_ref, w1_ref, isd_ref, hw_ref):
  deg = d0_ref[...] + d1_ref[...] + 1.0
  isd = lax.rsqrt(deg)
  isd_ref[...] = isd
  hw = jnp.dot(x_ref[...], w1_ref[...], preferred_element_type=jnp.float32)
  hw_ref[...] = hw * isd


def _tc1(d0, d1, x, w1p):
  return pl.pallas_call(
      _tc1_body,
      grid=(_GRID,),
      in_specs=[_row_spec(1), _row_spec(1), _row_spec(D), _full_spec((D, 24))],
      out_specs=[_row_spec(1), _row_spec(24)],
      out_shape=[
          jax.ShapeDtypeStruct((N, 1), jnp.float32),
          jax.ShapeDtypeStruct((N, 24), jnp.float32),
      ],
  )(d0, d1, x, w1p)


def _tc2_body(s0_ref, s1_ref, hw_ref, isd_ref, out_ref):
  isd = isd_ref[...]
  agg = isd * (s0_ref[...] + s1_ref[...] + hw_ref[...])
  out_ref[...] = jnp.maximum(agg, 0.0) * isd


def _tc2(s0, s1, hw, isd):
  return pl.pallas_call(
      _tc2_body,
      grid=(_GRID,),
      in_specs=[_row_spec(24), _row_spec(24), _row_spec(24), _row_spec(1)],
      out_specs=_row_spec(24),
      out_shape=jax.ShapeDtypeStruct((N, 24), jnp.float32),
  )(s0, s1, hw, isd)


def _tc3_body(s0_ref, s1_ref, h1p_ref, isd_ref, w2_ref, w3_ref, out_ref):
  isd = isd_ref[...]
  t = isd * (s0_ref[...] + s1_ref[...] + h1p_ref[...])
  h2 = jnp.maximum(
      jnp.dot(t, w2_ref[...], preferred_element_type=jnp.float32), 0.0)
  hw3 = jnp.dot(h2, w3_ref[...], preferred_element_type=jnp.float32)
  out_ref[...] = hw3 * isd


def _tc3(s0, s1, h1p, isd, w2p, w3p):
  return pl.pallas_call(
      _tc3_body,
      grid=(_GRID,),
      in_specs=[
          _row_spec(24), _row_spec(24), _row_spec(24), _row_spec(1),
          _full_spec((24, 40)), _full_spec((40, 8)),
      ],
      out_specs=_row_spec(8),
      out_shape=jax.ShapeDtypeStruct((N, 8), jnp.float32),
  )(s0, s1, h1p, isd, w2p, w3p)


def _tc4_body(s0_ref, s1_ref, hw_ref, isd_ref, out_ref):
  out_ref[...] = isd_ref[...] * (s0_ref[...] + s1_ref[...] + hw_ref[...])


def _tc4(s0, s1, hw, isd):
  return pl.pallas_call(
      _tc4_body,
      grid=(_GRID,),
      in_specs=[_row_spec(8), _row_spec(8), _row_spec(8), _row_spec(1)],
      out_specs=_row_spec(8),
      out_shape=jax.ShapeDtypeStruct((N, 8), jnp.float32),
  )(s0, s1, hw, isd)


_sc_edge24 = _make_sc_edge(24)
_sc_edge8 = _make_sc_edge(8)


@jax.jit
def kernel(x, edge_index, W1, W2, W3):
  pad = EPAD - E
  src2d = jnp.concatenate(
      [edge_index[0], jnp.zeros((pad,), jnp.int32)]).reshape(NGP, G)
  dst2d = jnp.concatenate(
      [edge_index[1], jnp.full((pad,), N, jnp.int32)]).reshape(NGP, G)
  w1p = jnp.pad(W1, ((0, 0), (0, 4)))
  w2p = jnp.pad(W2, ((0, 4), (0, 0)))
  w3p = jnp.pad(W3, ((0, 0), (0, 6)))
  zeros8 = jnp.zeros((RPS, 8), jnp.float32)
  zeros24 = jnp.zeros((RPS, 24), jnp.float32)

  degp = _sc_degree(dst2d, zeros8)
  d0 = degp[:N, :1]
  d1 = degp[NPAD:NPAD + N, :1]

  isd, hw1p = _tc1(d0, d1, x, w1p)
  s1 = _sc_edge24(hw1p, src2d, dst2d, zeros24)
  h1p = _tc2(s1[:N], s1[NPAD:NPAD + N], hw1p, isd)
  s2 = _sc_edge24(h1p, src2d, dst2d, zeros24)
  hw3p = _tc3(s2[:N], s2[NPAD:NPAD + N], h1p, isd, w2p, w3p)
  s3 = _sc_edge8(hw3p, src2d, dst2d, zeros8)
  out8 = _tc4(s3[:N], s3[NPAD:NPAD + N], hw3p, isd)
  return out8[:, :2]
